# baseline (speedup 1.0000x reference)
import functools

import jax
import jax.numpy as jnp
from jax import lax
from jax.experimental import pallas as pl
from jax.experimental.pallas import tpu as pltpu
from jax.experimental.pallas import tpu_sc as plsc

B = 16384
EMBED = 64

NC, NS = 2, 16
NW = NC * NS
B_PER_W = B // NW


@functools.cache
def _make_sc_gather():
    mesh = plsc.VectorSubcoreMesh(
        core_axis_name="c", subcore_axis_name="s",
        num_cores=NC, num_subcores=NS)

    @functools.partial(
        pl.kernel,
        mesh=mesh,
        compiler_params=pltpu.CompilerParams(use_tc_tiling_on_sc=False),
        out_type=jax.ShapeDtypeStruct((B, 2 * EMBED), jnp.bfloat16),
        scratch_types=[
            pltpu.VMEM((B_PER_W,), jnp.int32),
            pltpu.VMEM((B_PER_W,), jnp.int32),
            pltpu.VMEM((B_PER_W, 2 * EMBED), jnp.bfloat16),
            pltpu.SemaphoreType.DMA,
        ],
    )
    def _sc_gather(user_idx, item_idx, user_tab, item_tab,
                   out, uidx_v, iidx_v, rowsbuf, sem):
        wid = lax.axis_index("s") * NC + lax.axis_index("c")
        base = wid * B_PER_W
        pltpu.sync_copy(user_idx.at[pl.ds(base, B_PER_W)], uidx_v)
        pltpu.sync_copy(item_idx.at[pl.ds(base, B_PER_W)], iidx_v)

        def body(j, _):
            k0 = j * 16
            uv = uidx_v[pl.ds(k0, 16)]
            iv = iidx_v[pl.ds(k0, 16)]
            for l in range(16):
                pltpu.async_copy(
                    user_tab.at[uv[l]],
                    rowsbuf.at[k0 + l, pl.ds(0, EMBED)], sem)
                pltpu.async_copy(
                    item_tab.at[iv[l]],
                    rowsbuf.at[k0 + l, pl.ds(EMBED, EMBED)], sem)
            return 0

        lax.fori_loop(0, B_PER_W // 16, body, 0)
        pltpu.make_async_copy(out.at[pl.ds(0, B_PER_W)], rowsbuf, sem).wait()
        pltpu.sync_copy(rowsbuf, out.at[pl.ds(base, B_PER_W)])

    return _sc_gather


def _mlp_body(x_ref, w1_ref, b1_ref, w2_ref, b2_ref, w3_ref, b3_ref, out_ref):
    h1 = jnp.dot(x_ref[...], w1_ref[...], preferred_element_type=jnp.float32)
    h1 = jnp.maximum(h1 + b1_ref[...], 0.0)
    h2 = jnp.dot(h1, w2_ref[...], preferred_element_type=jnp.float32)
    h2 = jnp.maximum(h2 + b2_ref[...], 0.0)
    logit = jnp.sum(h2 * w3_ref[...], axis=1, keepdims=True) + b3_ref[...]
    out_ref[...] = 5.0 / (1.0 + jnp.exp(-logit))


def _tc_mlp(x, W1, b1, W2, b2, W3, b3):
    blk = 2048
    grid = (B // blk,)
    full = lambda shape: pl.BlockSpec(shape, lambda i: (0, 0))
    return pl.pallas_call(
        _mlp_body,
        grid=grid,
        in_specs=[
            pl.BlockSpec((blk, 2 * EMBED), lambda i: (i, 0)),
            full((2 * EMBED, 128)),
            full((1, 128)),
            full((128, 64)),
            full((1, 64)),
            full((1, 64)),
            full((1, 1)),
        ],
        out_specs=pl.BlockSpec((blk, 1), lambda i: (i, 0)),
        out_shape=jax.ShapeDtypeStruct((B, 1), jnp.float32),
    )(x, W1, b1.reshape(1, -1), W2, b2.reshape(1, -1),
      W3.reshape(1, -1), b3.reshape(1, 1))


def kernel(user_input, item_input, user_table, item_table, W1, b1, W2, b2, W3, b3):
    x = _make_sc_gather()(
        user_input, item_input,
        user_table.astype(jnp.bfloat16), item_table.astype(jnp.bfloat16))
    return _tc_mlp(x, W1, b1, W2, b2, W3, b3)


# speedup vs baseline: 1.8017x; 1.8017x over previous
import functools

import jax
import jax.numpy as jnp
from jax import lax
from jax.experimental import pallas as pl
from jax.experimental.pallas import tpu as pltpu
from jax.experimental.pallas import tpu_sc as plsc

B = 16384
EMBED = 64
PAIR = 2 * EMBED

TBLK = 3200
HALFB = TBLK // 2
NTB = 313
NPACK = NTB * HALFB

NC, NS = 2, 16
NW = NC * NS
B_PER_W = B // NW


def _pack_body(x_ref, o_ref):
    x = x_ref[...]
    t1 = jnp.transpose(x[:, :HALFB])
    t2 = jnp.transpose(x[:, HALFB:])
    o_ref[...] = jnp.concatenate([t1, t2], axis=1)


def _pack_table(tab_t):
    return pl.pallas_call(
        _pack_body,
        grid=(NTB,),
        in_specs=[pl.BlockSpec((EMBED, TBLK), lambda i: (0, i))],
        out_specs=pl.BlockSpec((HALFB, PAIR), lambda i: (i, 0)),
        out_shape=jax.ShapeDtypeStruct((NPACK, PAIR), jnp.float32),
        compiler_params=pltpu.CompilerParams(
            dimension_semantics=("arbitrary",)),
    )(tab_t)


@functools.cache
def _make_sc_gather():
    mesh = plsc.VectorSubcoreMesh(
        core_axis_name="c", subcore_axis_name="s",
        num_cores=NC, num_subcores=NS)

    @functools.partial(
        pl.kernel,
        mesh=mesh,
        out_type=jax.ShapeDtypeStruct((B, 2 * PAIR), jnp.float32),
        scratch_types=[
            pltpu.VMEM((B_PER_W,), jnp.int32),
            pltpu.VMEM((B_PER_W,), jnp.int32),
            pltpu.VMEM((B_PER_W // 2, 2 * PAIR), jnp.float32),
            pltpu.SemaphoreType.DMA,
        ],
    )
    def _sc_gather(user_idx, item_idx, user_tab, item_tab,
                   out, uidx_v, iidx_v, rowsbuf, sem):
        wid = lax.axis_index("s") * NC + lax.axis_index("c")
        base = wid * B_PER_W
        pltpu.sync_copy(user_idx.at[pl.ds(base, B_PER_W)], uidx_v)
        pltpu.sync_copy(item_idx.at[pl.ds(base, B_PER_W)], iidx_v)

        half = B_PER_W // 2
        for h in range(2):
            def body(j, _, h=h):
                k0 = h * half + j * 16
                uv = uidx_v[pl.ds(k0, 16)]
                iv = iidx_v[pl.ds(k0, 16)]
                for l in range(16):
                    pltpu.async_copy(
                        user_tab.at[uv[l]],
                        rowsbuf.at[j * 16 + l, pl.ds(0, PAIR)], sem)
                    pltpu.async_copy(
                        item_tab.at[iv[l]],
                        rowsbuf.at[j * 16 + l, pl.ds(PAIR, PAIR)], sem)
                return 0

            lax.fori_loop(0, half // 16, body, 0)
            pltpu.make_async_copy(out.at[pl.ds(0, half)], rowsbuf, sem).wait()
            pltpu.sync_copy(rowsbuf, out.at[pl.ds(base + h * half, half)])

    return _sc_gather


def _mlp_body(x_ref, up_ref, ip_ref, w1u_ref, w1v_ref, b1_ref, w2_ref,
              b2_ref, w3_ref, b3_ref, out_ref):
    x = x_ref[...]
    u = jnp.where(up_ref[...] > 0, x[:, EMBED:PAIR], x[:, 0:EMBED])
    v = jnp.where(ip_ref[...] > 0, x[:, PAIR + EMBED:], x[:, PAIR:PAIR + EMBED])
    h1 = jnp.dot(u, w1u_ref[...], preferred_element_type=jnp.float32)
    h1 += jnp.dot(v, w1v_ref[...], preferred_element_type=jnp.float32)
    h1 = jnp.maximum(h1 + b1_ref[...], 0.0)
    h2 = jnp.dot(h1, w2_ref[...], preferred_element_type=jnp.float32)
    h2 = jnp.maximum(h2 + b2_ref[...], 0.0)
    logit = jnp.sum(h2 * w3_ref[...], axis=1, keepdims=True) + b3_ref[...]
    out_ref[...] = 5.0 / (1.0 + jnp.exp(-logit))


def _tc_mlp(x, upar, ipar, W1, b1, W2, b2, W3, b3):
    blk = 2048
    grid = (B // blk,)
    full = lambda shape: pl.BlockSpec(shape, lambda i: (0, 0))
    return pl.pallas_call(
        _mlp_body,
        grid=grid,
        in_specs=[
            pl.BlockSpec((blk, 2 * PAIR), lambda i: (i, 0)),
            pl.BlockSpec((blk, 1), lambda i: (i, 0)),
            pl.BlockSpec((blk, 1), lambda i: (i, 0)),
            full((EMBED, 128)),
            full((EMBED, 128)),
            full((1, 128)),
            full((128, 64)),
            full((1, 64)),
            full((1, 64)),
            full((1, 1)),
        ],
        out_specs=pl.BlockSpec((blk, 1), lambda i: (i, 0)),
        out_shape=jax.ShapeDtypeStruct((B, 1), jnp.float32),
    )(x, upar, ipar, W1[:EMBED], W1[EMBED:], b1.reshape(1, -1),
      W2, b2.reshape(1, -1), W3.reshape(1, -1), b3.reshape(1, 1))


def _split_idx(idx, n):
    b = idx // TBLK
    start = jnp.minimum(b * TBLK, n - TBLK)
    w = idx - start
    pr = b * HALFB + w % HALFB
    half = (w // HALFB).astype(jnp.int32).reshape(B, 1)
    return pr, half


def kernel(user_input, item_input, user_table, item_table, W1, b1, W2, b2, W3, b3):
    user_packed = _pack_table(user_table.T)
    item_packed = _pack_table(item_table.T)
    upr, upar = _split_idx(user_input, user_table.shape[0])
    ipr, ipar = _split_idx(item_input, item_table.shape[0])
    x = _make_sc_gather()(upr, ipr, user_packed, item_packed)
    return _tc_mlp(x, upar, ipar, W1, b1, W2, b2, W3, b3)


# speedup vs baseline: 2.0715x; 1.1497x over previous
import functools

import jax
import jax.numpy as jnp
from jax import lax
from jax.experimental import pallas as pl
from jax.experimental.pallas import tpu as pltpu
from jax.experimental.pallas import tpu_sc as plsc

B = 16384
EMBED = 64

NC, NS = 2, 16
NW = NC * NS
B_PER_W = B // NW


@functools.cache
def _make_sc_gather():
    mesh = plsc.VectorSubcoreMesh(
        core_axis_name="c", subcore_axis_name="s",
        num_cores=NC, num_subcores=NS)

    @functools.partial(
        pl.kernel,
        mesh=mesh,
        out_type=jax.ShapeDtypeStruct((B, 2 * EMBED), jnp.float32),
        scratch_types=[
            pltpu.VMEM((B_PER_W,), jnp.int32),
            pltpu.VMEM((B_PER_W,), jnp.int32),
            pltpu.VMEM((B_PER_W, 2 * EMBED), jnp.float32),
            pltpu.SemaphoreType.DMA,
        ],
    )
    def _sc_gather(user_idx, item_idx, user_tab, item_tab,
                   out, uidx_v, iidx_v, rowsbuf, sem):
        wid = lax.axis_index("s") * NC + lax.axis_index("c")
        base = wid * B_PER_W
        pltpu.sync_copy(user_idx.at[pl.ds(base, B_PER_W)], uidx_v)
        pltpu.sync_copy(item_idx.at[pl.ds(base, B_PER_W)], iidx_v)

        def body(j, _):
            k0 = j * 16
            uv = uidx_v[pl.ds(k0, 16)]
            iv = iidx_v[pl.ds(k0, 16)]
            for l in range(16):
                pltpu.async_copy(
                    user_tab.at[uv[l]],
                    rowsbuf.at[k0 + l, pl.ds(0, EMBED)], sem)
                pltpu.async_copy(
                    item_tab.at[iv[l]],
                    rowsbuf.at[k0 + l, pl.ds(EMBED, EMBED)], sem)
            return 0

        lax.fori_loop(0, B_PER_W // 16, body, 0)
        pltpu.make_async_copy(out.at[pl.ds(0, B_PER_W)], rowsbuf, sem).wait()
        pltpu.sync_copy(rowsbuf, out.at[pl.ds(base, B_PER_W)])

    return _sc_gather


def _mlp_body(x_ref, w1_ref, b1_ref, w2_ref, b2_ref, w3_ref, b3_ref, out_ref):
    h1 = jnp.dot(x_ref[...], w1_ref[...], preferred_element_type=jnp.float32)
    h1 = jnp.maximum(h1 + b1_ref[...], 0.0)
    h2 = jnp.dot(h1, w2_ref[...], preferred_element_type=jnp.float32)
    h2 = jnp.maximum(h2 + b2_ref[...], 0.0)
    logit = jnp.sum(h2 * w3_ref[...], axis=1, keepdims=True) + b3_ref[...]
    out_ref[...] = 5.0 / (1.0 + jnp.exp(-logit))


def _tc_mlp(x, W1, b1, W2, b2, W3, b3):
    blk = 2048
    grid = (B // blk,)
    full = lambda shape: pl.BlockSpec(shape, lambda i: (0, 0))
    return pl.pallas_call(
        _mlp_body,
        grid=grid,
        in_specs=[
            pl.BlockSpec((blk, 2 * EMBED), lambda i: (i, 0)),
            full((2 * EMBED, 128)),
            full((1, 128)),
            full((128, 64)),
            full((1, 64)),
            full((1, 64)),
            full((1, 1)),
        ],
        out_specs=pl.BlockSpec((blk, 1), lambda i: (i, 0)),
        out_shape=jax.ShapeDtypeStruct((B, 1), jnp.float32),
    )(x, W1, b1.reshape(1, -1), W2, b2.reshape(1, -1),
      W3.reshape(1, -1), b3.reshape(1, 1))


def kernel(user_input, item_input, user_table, item_table, W1, b1, W2, b2, W3, b3):
    x = _make_sc_gather()(user_input, item_input, user_table, item_table)
    return _tc_mlp(x, W1, b1, W2, b2, W3, b3)


# speedup vs baseline: 2.2812x; 1.1013x over previous
import functools

import jax
import jax.numpy as jnp
from jax import lax
from jax.experimental import pallas as pl
from jax.experimental.pallas import tpu as pltpu
from jax.experimental.pallas import tpu_sc as plsc

B = 16384
EMBED = 64
PAIR = 2 * EMBED

TBLK = 3328
HALFB = TBLK // 2
NTB = 301
NPACK = NTB * HALFB

NC, NS = 2, 16
NW = NC * NS
B_PER_W = B // NW


def _pack_body(xa_ref, xb_ref, o_ref):
    o_ref[...] = jnp.transpose(
        jnp.concatenate([xa_ref[...], xb_ref[...]], axis=0))


def _pack_table(tab_t):
    n = tab_t.shape[1]
    last = n // HALFB
    return pl.pallas_call(
        _pack_body,
        grid=(NTB,),
        in_specs=[
            pl.BlockSpec((EMBED, HALFB),
                         lambda i: (0, jnp.minimum(2 * i, last - 1))),
            pl.BlockSpec((EMBED, HALFB),
                         lambda i: (0, jnp.minimum(2 * i + 1, last))),
        ],
        out_specs=pl.BlockSpec((HALFB, PAIR), lambda i: (i, 0)),
        out_shape=jax.ShapeDtypeStruct((NPACK, PAIR), jnp.float32),
        compiler_params=pltpu.CompilerParams(
            dimension_semantics=("arbitrary",)),
    )(tab_t, tab_t)


@functools.cache
def _make_sc_gather():
    mesh = plsc.VectorSubcoreMesh(
        core_axis_name="c", subcore_axis_name="s",
        num_cores=NC, num_subcores=NS)

    @functools.partial(
        pl.kernel,
        mesh=mesh,
        out_type=jax.ShapeDtypeStruct((B, 2 * PAIR), jnp.float32),
        scratch_types=[
            pltpu.VMEM((B_PER_W,), jnp.int32),
            pltpu.VMEM((B_PER_W,), jnp.int32),
            pltpu.VMEM((B_PER_W // 2, 2 * PAIR), jnp.float32),
            pltpu.SemaphoreType.DMA,
        ],
    )
    def _sc_gather(user_idx, item_idx, user_tab, item_tab,
                   out, uidx_v, iidx_v, rowsbuf, sem):
        wid = lax.axis_index("s") * NC + lax.axis_index("c")
        base = wid * B_PER_W
        pltpu.sync_copy(user_idx.at[pl.ds(base, B_PER_W)], uidx_v)
        pltpu.sync_copy(item_idx.at[pl.ds(base, B_PER_W)], iidx_v)

        half = B_PER_W // 2
        for h in range(2):
            def body(j, _, h=h):
                k0 = h * half + j * 16
                uv = uidx_v[pl.ds(k0, 16)]
                iv = iidx_v[pl.ds(k0, 16)]
                for l in range(16):
                    pltpu.async_copy(
                        user_tab.at[uv[l]],
                        rowsbuf.at[j * 16 + l, pl.ds(0, PAIR)], sem)
                    pltpu.async_copy(
                        item_tab.at[iv[l]],
                        rowsbuf.at[j * 16 + l, pl.ds(PAIR, PAIR)], sem)
                return 0

            lax.fori_loop(0, half // 16, body, 0)
            pltpu.make_async_copy(out.at[pl.ds(0, half)], rowsbuf, sem).wait()
            pltpu.sync_copy(rowsbuf, out.at[pl.ds(base + h * half, half)])

    return _sc_gather


def _mlp_body(x_ref, up_ref, ip_ref, w1u_ref, w1v_ref, b1_ref, w2_ref,
              b2_ref, w3_ref, b3_ref, out_ref):
    x = x_ref[...]
    u = jnp.where(up_ref[...] > 0, x[:, EMBED:PAIR], x[:, 0:EMBED])
    v = jnp.where(ip_ref[...] > 0, x[:, PAIR + EMBED:], x[:, PAIR:PAIR + EMBED])
    h1 = jnp.dot(u, w1u_ref[...], preferred_element_type=jnp.float32)
    h1 += jnp.dot(v, w1v_ref[...], preferred_element_type=jnp.float32)
    h1 = jnp.maximum(h1 + b1_ref[...], 0.0)
    h2 = jnp.dot(h1, w2_ref[...], preferred_element_type=jnp.float32)
    h2 = jnp.maximum(h2 + b2_ref[...], 0.0)
    logit = jnp.sum(h2 * w3_ref[...], axis=1, keepdims=True) + b3_ref[...]
    out_ref[...] = 5.0 / (1.0 + jnp.exp(-logit))


def _tc_mlp(x, upar, ipar, W1, b1, W2, b2, W3, b3):
    blk = 2048
    grid = (B // blk,)
    full = lambda shape: pl.BlockSpec(shape, lambda i: (0, 0))
    return pl.pallas_call(
        _mlp_body,
        grid=grid,
        in_specs=[
            pl.BlockSpec((blk, 2 * PAIR), lambda i: (i, 0)),
            pl.BlockSpec((blk, 1), lambda i: (i, 0)),
            pl.BlockSpec((blk, 1), lambda i: (i, 0)),
            full((EMBED, 128)),
            full((EMBED, 128)),
            full((1, 128)),
            full((128, 64)),
            full((1, 64)),
            full((1, 64)),
            full((1, 1)),
        ],
        out_specs=pl.BlockSpec((blk, 1), lambda i: (i, 0)),
        out_shape=jax.ShapeDtypeStruct((B, 1), jnp.float32),
    )(x, upar, ipar, W1[:EMBED], W1[EMBED:], b1.reshape(1, -1),
      W2, b2.reshape(1, -1), W3.reshape(1, -1), b3.reshape(1, 1))


def _split_idx(idx, n):
    b = idx // TBLK
    start = jnp.minimum(b * TBLK, (n // HALFB - 1) * HALFB)
    w = idx - start
    pr = b * HALFB + w % HALFB
    half = (w // HALFB).astype(jnp.int32).reshape(B, 1)
    return pr, half


def kernel(user_input, item_input, user_table, item_table, W1, b1, W2, b2, W3, b3):
    user_packed = _pack_table(user_table.T)
    item_packed = _pack_table(item_table.T)
    upr, upar = _split_idx(user_input, user_table.shape[0])
    ipr, ipar = _split_idx(item_input, item_table.shape[0])
    x = _make_sc_gather()(upr, ipr, user_packed, item_packed)
    return _tc_mlp(x, upar, ipar, W1, b1, W2, b2, W3, b3)


# speedup vs baseline: 3.0497x; 1.3369x over previous
import functools

import jax
import jax.numpy as jnp
from jax import lax
from jax.experimental import pallas as pl
from jax.experimental.pallas import tpu as pltpu
from jax.experimental.pallas import tpu_sc as plsc

B = 16384
EMBED = 64
PAIR = 2 * EMBED

TBLK = 6656
HALFB = TBLK // 2
NTB = 151
NPACK = NTB * HALFB

NC, NS = 2, 16
NW = NC * NS
B_PER_W = B // NW


def _pack_body(xa_ref, xb_ref, o_ref):
    o_ref[...] = jnp.transpose(
        jnp.concatenate([xa_ref[...], xb_ref[...]], axis=0))


def _pack_table(tab_t):
    n = tab_t.shape[1]
    last = n // HALFB
    return pl.pallas_call(
        _pack_body,
        grid=(NTB,),
        in_specs=[
            pl.BlockSpec((EMBED, HALFB),
                         lambda i: (0, jnp.minimum(2 * i, last - 1))),
            pl.BlockSpec((EMBED, HALFB),
                         lambda i: (0, jnp.minimum(2 * i + 1, last))),
        ],
        out_specs=pl.BlockSpec((HALFB, PAIR), lambda i: (i, 0)),
        out_shape=jax.ShapeDtypeStruct((NPACK, PAIR), jnp.float32),
        compiler_params=pltpu.CompilerParams(
            dimension_semantics=("arbitrary",)),
    )(tab_t, tab_t)


@functools.cache
def _make_sc_gather():
    mesh = plsc.VectorSubcoreMesh(
        core_axis_name="c", subcore_axis_name="s",
        num_cores=NC, num_subcores=NS)

    @functools.partial(
        pl.kernel,
        mesh=mesh,
        out_type=jax.ShapeDtypeStruct((B, 2 * PAIR), jnp.float32),
        scratch_types=[
            pltpu.VMEM((B_PER_W,), jnp.int32),
            pltpu.VMEM((B_PER_W,), jnp.int32),
            pltpu.VMEM((B_PER_W // 2, 2 * PAIR), jnp.float32),
            pltpu.SemaphoreType.DMA,
        ],
    )
    def _sc_gather(user_idx, item_idx, user_tab, item_tab,
                   out, uidx_v, iidx_v, rowsbuf, sem):
        wid = lax.axis_index("s") * NC + lax.axis_index("c")
        base = wid * B_PER_W
        pltpu.sync_copy(user_idx.at[pl.ds(base, B_PER_W)], uidx_v)
        pltpu.sync_copy(item_idx.at[pl.ds(base, B_PER_W)], iidx_v)

        half = B_PER_W // 2
        for h in range(2):
            def body(j, _, h=h):
                k0 = h * half + j * 16
                uv = uidx_v[pl.ds(k0, 16)]
                iv = iidx_v[pl.ds(k0, 16)]
                for l in range(16):
                    pltpu.async_copy(
                        user_tab.at[uv[l]],
                        rowsbuf.at[j * 16 + l, pl.ds(0, PAIR)], sem)
                    pltpu.async_copy(
                        item_tab.at[iv[l]],
                        rowsbuf.at[j * 16 + l, pl.ds(PAIR, PAIR)], sem)
                return 0

            lax.fori_loop(0, half // 16, body, 0)
            pltpu.make_async_copy(out.at[pl.ds(0, half)], rowsbuf, sem).wait()
            pltpu.sync_copy(rowsbuf, out.at[pl.ds(base + h * half, half)])

    return _sc_gather


def _mlp_body(x_ref, up_ref, ip_ref, w1u_ref, w1v_ref, b1_ref, w2_ref,
              b2_ref, w3_ref, b3_ref, out_ref):
    x = x_ref[...]
    u = jnp.where(up_ref[...] > 0, x[:, EMBED:PAIR], x[:, 0:EMBED])
    v = jnp.where(ip_ref[...] > 0, x[:, PAIR + EMBED:], x[:, PAIR:PAIR + EMBED])
    h1 = jnp.dot(u, w1u_ref[...], preferred_element_type=jnp.float32)
    h1 += jnp.dot(v, w1v_ref[...], preferred_element_type=jnp.float32)
    h1 = jnp.maximum(h1 + b1_ref[...], 0.0)
    h2 = jnp.dot(h1, w2_ref[...], preferred_element_type=jnp.float32)
    h2 = jnp.maximum(h2 + b2_ref[...], 0.0)
    logit = jnp.sum(h2 * w3_ref[...], axis=1, keepdims=True) + b3_ref[...]
    out_ref[...] = 5.0 / (1.0 + jnp.exp(-logit))


def _tc_mlp(x, upar, ipar, W1, b1, W2, b2, W3, b3):
    blk = 2048
    grid = (B // blk,)
    full = lambda shape: pl.BlockSpec(shape, lambda i: (0, 0))
    return pl.pallas_call(
        _mlp_body,
        grid=grid,
        in_specs=[
            pl.BlockSpec((blk, 2 * PAIR), lambda i: (i, 0)),
            pl.BlockSpec((blk, 1), lambda i: (i, 0)),
            pl.BlockSpec((blk, 1), lambda i: (i, 0)),
            full((EMBED, 128)),
            full((EMBED, 128)),
            full((1, 128)),
            full((128, 64)),
            full((1, 64)),
            full((1, 64)),
            full((1, 1)),
        ],
        out_specs=pl.BlockSpec((blk, 1), lambda i: (i, 0)),
        out_shape=jax.ShapeDtypeStruct((B, 1), jnp.float32),
    )(x, upar, ipar, W1[:EMBED], W1[EMBED:], b1.reshape(1, -1),
      W2, b2.reshape(1, -1), W3.reshape(1, -1), b3.reshape(1, 1))


def _split_idx(idx, n):
    b = idx // TBLK
    start = jnp.minimum(b * TBLK, (n // HALFB - 1) * HALFB)
    w = idx - start
    pr = b * HALFB + w % HALFB
    half = (w // HALFB).astype(jnp.int32).reshape(B, 1)
    return pr, half


def kernel(user_input, item_input, user_table, item_table, W1, b1, W2, b2, W3, b3):
    user_packed = _pack_table(user_table.T)
    item_packed = _pack_table(item_table.T)
    upr, upar = _split_idx(user_input, user_table.shape[0])
    ipr, ipar = _split_idx(item_input, item_table.shape[0])
    x = _make_sc_gather()(upr, ipr, user_packed, item_packed)
    return _tc_mlp(x, upar, ipar, W1, b1, W2, b2, W3, b3)


# speedup vs baseline: 3.6630x; 1.2011x over previous
import functools

import jax
import jax.numpy as jnp
from jax import lax
from jax.experimental import pallas as pl
from jax.experimental.pallas import tpu as pltpu
from jax.experimental.pallas import tpu_sc as plsc

B = 16384
EMBED = 64
PAIR = 2 * EMBED

TBLK = 13312
HALFB = TBLK // 2
NTB = 76
NPACK = NTB * HALFB

NC, NS = 2, 16
NW = NC * NS
B_PER_W = B // NW


def _pack_body(xa_ref, xb_ref, o_ref):
    o_ref[...] = jnp.transpose(
        jnp.concatenate([xa_ref[...], xb_ref[...]], axis=0))


def _pack_table(tab_t):
    n = tab_t.shape[1]
    last = n // HALFB
    return pl.pallas_call(
        _pack_body,
        grid=(NTB,),
        in_specs=[
            pl.BlockSpec((EMBED, HALFB),
                         lambda i: (0, jnp.minimum(2 * i, last - 1))),
            pl.BlockSpec((EMBED, HALFB),
                         lambda i: (0, jnp.minimum(2 * i + 1, last))),
        ],
        out_specs=pl.BlockSpec((HALFB, PAIR), lambda i: (i, 0)),
        out_shape=jax.ShapeDtypeStruct((NPACK, PAIR), jnp.float32),
        compiler_params=pltpu.CompilerParams(
            dimension_semantics=("arbitrary",)),
    )(tab_t, tab_t)


@functools.cache
def _make_sc_gather():
    mesh = plsc.VectorSubcoreMesh(
        core_axis_name="c", subcore_axis_name="s",
        num_cores=NC, num_subcores=NS)

    @functools.partial(
        pl.kernel,
        mesh=mesh,
        out_type=jax.ShapeDtypeStruct((B, 2 * PAIR), jnp.float32),
        scratch_types=[
            pltpu.VMEM((B_PER_W,), jnp.int32),
            pltpu.VMEM((B_PER_W,), jnp.int32),
            pltpu.VMEM((B_PER_W // 2, 2 * PAIR), jnp.float32),
            pltpu.SemaphoreType.DMA,
        ],
    )
    def _sc_gather(user_idx, item_idx, user_tab, item_tab,
                   out, uidx_v, iidx_v, rowsbuf, sem):
        wid = lax.axis_index("s") * NC + lax.axis_index("c")
        base = wid * B_PER_W
        pltpu.sync_copy(user_idx.at[pl.ds(base, B_PER_W)], uidx_v)
        pltpu.sync_copy(item_idx.at[pl.ds(base, B_PER_W)], iidx_v)

        half = B_PER_W // 2
        for h in range(2):
            def body(j, _, h=h):
                k0 = h * half + j * 16
                uv = uidx_v[pl.ds(k0, 16)]
                iv = iidx_v[pl.ds(k0, 16)]
                for l in range(16):
                    pltpu.async_copy(
                        user_tab.at[uv[l]],
                        rowsbuf.at[j * 16 + l, pl.ds(0, PAIR)], sem)
                    pltpu.async_copy(
                        item_tab.at[iv[l]],
                        rowsbuf.at[j * 16 + l, pl.ds(PAIR, PAIR)], sem)
                return 0

            lax.fori_loop(0, half // 16, body, 0)
            pltpu.make_async_copy(out.at[pl.ds(0, half)], rowsbuf, sem).wait()
            pltpu.sync_copy(rowsbuf, out.at[pl.ds(base + h * half, half)])

    return _sc_gather


def _mlp_body(x_ref, up_ref, ip_ref, w1u_ref, w1v_ref, b1_ref, w2_ref,
              b2_ref, w3_ref, b3_ref, out_ref):
    x = x_ref[...]
    u = jnp.where(up_ref[...] > 0, x[:, EMBED:PAIR], x[:, 0:EMBED])
    v = jnp.where(ip_ref[...] > 0, x[:, PAIR + EMBED:], x[:, PAIR:PAIR + EMBED])
    h1 = jnp.dot(u, w1u_ref[...], preferred_element_type=jnp.float32)
    h1 += jnp.dot(v, w1v_ref[...], preferred_element_type=jnp.float32)
    h1 = jnp.maximum(h1 + b1_ref[...], 0.0)
    h2 = jnp.dot(h1, w2_ref[...], preferred_element_type=jnp.float32)
    h2 = jnp.maximum(h2 + b2_ref[...], 0.0)
    logit = jnp.sum(h2 * w3_ref[...], axis=1, keepdims=True) + b3_ref[...]
    out_ref[...] = 5.0 / (1.0 + jnp.exp(-logit))


def _tc_mlp(x, upar, ipar, W1, b1, W2, b2, W3, b3):
    blk = 2048
    grid = (B // blk,)
    full = lambda shape: pl.BlockSpec(shape, lambda i: (0, 0))
    return pl.pallas_call(
        _mlp_body,
        grid=grid,
        in_specs=[
            pl.BlockSpec((blk, 2 * PAIR), lambda i: (i, 0)),
            pl.BlockSpec((blk, 1), lambda i: (i, 0)),
            pl.BlockSpec((blk, 1), lambda i: (i, 0)),
            full((EMBED, 128)),
            full((EMBED, 128)),
            full((1, 128)),
            full((128, 64)),
            full((1, 64)),
            full((1, 64)),
            full((1, 1)),
        ],
        out_specs=pl.BlockSpec((blk, 1), lambda i: (i, 0)),
        out_shape=jax.ShapeDtypeStruct((B, 1), jnp.float32),
    )(x, upar, ipar, W1[:EMBED], W1[EMBED:], b1.reshape(1, -1),
      W2, b2.reshape(1, -1), W3.reshape(1, -1), b3.reshape(1, 1))


def _split_idx(idx, n):
    b = idx // TBLK
    start = jnp.minimum(b * TBLK, (n // HALFB - 1) * HALFB)
    w = idx - start
    pr = b * HALFB + w % HALFB
    half = (w // HALFB).astype(jnp.int32).reshape(B, 1)
    return pr, half


def kernel(user_input, item_input, user_table, item_table, W1, b1, W2, b2, W3, b3):
    user_packed = _pack_table(user_table.T)
    item_packed = _pack_table(item_table.T)
    upr, upar = _split_idx(user_input, user_table.shape[0])
    ipr, ipar = _split_idx(item_input, item_table.shape[0])
    x = _make_sc_gather()(upr, ipr, user_packed, item_packed)
    return _tc_mlp(x, upar, ipar, W1, b1, W2, b2, W3, b3)


# speedup vs baseline: 3.8250x; 1.0442x over previous
import functools

import jax
import jax.numpy as jnp
from jax import lax
from jax.experimental import pallas as pl
from jax.experimental.pallas import tpu as pltpu
from jax.experimental.pallas import tpu_sc as plsc

B = 16384
EMBED = 64
PAIR = 2 * EMBED

TBLK = 26624
HALFB = TBLK // 2
NTB = 38
NPACK = NTB * HALFB

NC, NS = 2, 16
NW = NC * NS
B_PER_W = B // NW


def _pack_body(xa_ref, xb_ref, o_ref):
    o_ref[...] = jnp.transpose(
        jnp.concatenate([xa_ref[...], xb_ref[...]], axis=0))


def _pack_table(tab_t):
    n = tab_t.shape[1]
    last = n // HALFB
    return pl.pallas_call(
        _pack_body,
        grid=(NTB,),
        in_specs=[
            pl.BlockSpec((EMBED, HALFB),
                         lambda i: (0, jnp.minimum(2 * i, last - 1))),
            pl.BlockSpec((EMBED, HALFB),
                         lambda i: (0, jnp.minimum(2 * i + 1, last))),
        ],
        out_specs=pl.BlockSpec((HALFB, PAIR), lambda i: (i, 0)),
        out_shape=jax.ShapeDtypeStruct((NPACK, PAIR), jnp.float32),
        compiler_params=pltpu.CompilerParams(
            dimension_semantics=("arbitrary",)),
    )(tab_t, tab_t)


@functools.cache
def _make_sc_gather():
    mesh = plsc.VectorSubcoreMesh(
        core_axis_name="c", subcore_axis_name="s",
        num_cores=NC, num_subcores=NS)

    @functools.partial(
        pl.kernel,
        mesh=mesh,
        out_type=jax.ShapeDtypeStruct((B, 2 * PAIR), jnp.float32),
        scratch_types=[
            pltpu.VMEM((B_PER_W,), jnp.int32),
            pltpu.VMEM((B_PER_W,), jnp.int32),
            pltpu.VMEM((B_PER_W // 2, 2 * PAIR), jnp.float32),
            pltpu.SemaphoreType.DMA,
        ],
    )
    def _sc_gather(user_idx, item_idx, user_tab, item_tab,
                   out, uidx_v, iidx_v, rowsbuf, sem):
        wid = lax.axis_index("s") * NC + lax.axis_index("c")
        base = wid * B_PER_W
        pltpu.sync_copy(user_idx.at[pl.ds(base, B_PER_W)], uidx_v)
        pltpu.sync_copy(item_idx.at[pl.ds(base, B_PER_W)], iidx_v)

        half = B_PER_W // 2
        for h in range(2):
            def body(j, _, h=h):
                k0 = h * half + j * 16
                uv = uidx_v[pl.ds(k0, 16)]
                iv = iidx_v[pl.ds(k0, 16)]
                for l in range(16):
                    pltpu.async_copy(
                        user_tab.at[uv[l]],
                        rowsbuf.at[j * 16 + l, pl.ds(0, PAIR)], sem)
                    pltpu.async_copy(
                        item_tab.at[iv[l]],
                        rowsbuf.at[j * 16 + l, pl.ds(PAIR, PAIR)], sem)
                return 0

            lax.fori_loop(0, half // 16, body, 0)
            pltpu.make_async_copy(out.at[pl.ds(0, half)], rowsbuf, sem).wait()
            pltpu.sync_copy(rowsbuf, out.at[pl.ds(base + h * half, half)])

    return _sc_gather


def _mlp_body(x_ref, up_ref, ip_ref, w1u_ref, w1v_ref, b1_ref, w2_ref,
              b2_ref, w3_ref, b3_ref, out_ref):
    x = x_ref[...]
    u = jnp.where(up_ref[...] > 0, x[:, EMBED:PAIR], x[:, 0:EMBED])
    v = jnp.where(ip_ref[...] > 0, x[:, PAIR + EMBED:], x[:, PAIR:PAIR + EMBED])
    h1 = jnp.dot(u, w1u_ref[...], preferred_element_type=jnp.float32)
    h1 += jnp.dot(v, w1v_ref[...], preferred_element_type=jnp.float32)
    h1 = jnp.maximum(h1 + b1_ref[...], 0.0)
    h2 = jnp.dot(h1, w2_ref[...], preferred_element_type=jnp.float32)
    h2 = jnp.maximum(h2 + b2_ref[...], 0.0)
    logit = jnp.sum(h2 * w3_ref[...], axis=1, keepdims=True) + b3_ref[...]
    out_ref[...] = 5.0 / (1.0 + jnp.exp(-logit))


def _tc_mlp(x, upar, ipar, W1, b1, W2, b2, W3, b3):
    blk = 2048
    grid = (B // blk,)
    full = lambda shape: pl.BlockSpec(shape, lambda i: (0, 0))
    return pl.pallas_call(
        _mlp_body,
        grid=grid,
        in_specs=[
            pl.BlockSpec((blk, 2 * PAIR), lambda i: (i, 0)),
            pl.BlockSpec((blk, 1), lambda i: (i, 0)),
            pl.BlockSpec((blk, 1), lambda i: (i, 0)),
            full((EMBED, 128)),
            full((EMBED, 128)),
            full((1, 128)),
            full((128, 64)),
            full((1, 64)),
            full((1, 64)),
            full((1, 1)),
        ],
        out_specs=pl.BlockSpec((blk, 1), lambda i: (i, 0)),
        out_shape=jax.ShapeDtypeStruct((B, 1), jnp.float32),
    )(x, upar, ipar, W1[:EMBED], W1[EMBED:], b1.reshape(1, -1),
      W2, b2.reshape(1, -1), W3.reshape(1, -1), b3.reshape(1, 1))


def _split_idx(idx, n):
    b = idx // TBLK
    start = jnp.minimum(b * TBLK, (n // HALFB - 1) * HALFB)
    w = idx - start
    pr = b * HALFB + w % HALFB
    half = (w // HALFB).astype(jnp.int32).reshape(B, 1)
    return pr, half


def kernel(user_input, item_input, user_table, item_table, W1, b1, W2, b2, W3, b3):
    user_packed = _pack_table(user_table.T)
    item_packed = _pack_table(item_table.T)
    upr, upar = _split_idx(user_input, user_table.shape[0])
    ipr, ipar = _split_idx(item_input, item_table.shape[0])
    x = _make_sc_gather()(upr, ipr, user_packed, item_packed)
    return _tc_mlp(x, upar, ipar, W1, b1, W2, b2, W3, b3)
